# baseline (speedup 1.0000x reference)
import functools

import jax
import jax.numpy as jnp
from jax import lax
from jax.experimental import pallas as pl
from jax.experimental.pallas import tpu as pltpu
from jax.experimental.pallas import tpu_sc as plsc

N = 10000
E = 320000
D = 128
DE = 16
H = 256
L = 4
G = 64

HD = D // 2
CHUNK = 128
NCHUNK = E // CHUNK
NSUB = 16
CPT = 160
CBLK = 16
NCHPAD = CPT * NSUB
E_PAD = NCHPAD * CHUNK
NROW = N + 16
NPT_A = 632
NPT_ZL = NROW - 15 * NPT_A
NPT_SL = N - 15 * NPT_A


def _node_lin_body(x_ref, w_ref, b_ref, h_ref, tab_ref):
    h = jnp.dot(x_ref[...], w_ref[...], preferred_element_type=jnp.float32)
    h = h + b_ref[...]
    h_ref[...] = h
    tab_ref[:N, :] = h[:, :HD]
    tab_ref[N:, :] = h[:, HD:]


def _node_lin(x, w, b):
    return pl.pallas_call(
        _node_lin_body,
        out_shape=[
            jax.ShapeDtypeStruct((N, D), jnp.float32),
            jax.ShapeDtypeStruct((2 * N, HD), jnp.float32),
        ],
    )(x, w, b)


_BE = 16384


def _edge_lin_body(a_ref, w_ref, b_ref, out_ref):
    e = jnp.dot(a_ref[...], w_ref[...], preferred_element_type=jnp.float32)
    e = e + b_ref[...]
    out_ref[0] = e[:, :HD]
    out_ref[1] = e[:, HD:]


def _edge_lin(edge_attr, w, b):
    out = pl.pallas_call(
        _edge_lin_body,
        grid=(E_PAD // _BE,),
        in_specs=[
            pl.BlockSpec((_BE, DE), lambda i: (i, 0)),
            pl.BlockSpec((DE, D), lambda i: (0, 0)),
            pl.BlockSpec((1, D), lambda i: (0, 0)),
        ],
        out_specs=pl.BlockSpec((2, _BE, HD), lambda i: (0, i, 0)),
        out_shape=jax.ShapeDtypeStruct((2, E_PAD, HD), jnp.float32),
    )(edge_attr, w, b)
    return out.reshape(2 * E_PAD, HD)


def _idx_prep_body(ei_ref, src2_ref, dstp_ref):
    ei = ei_ref[...]
    src2_ref[:, :E] = jnp.concatenate([ei[0:1, :], ei[0:1, :] + N], axis=0)
    src2_ref[:, E:] = jnp.zeros((2, E_PAD - E), jnp.int32)
    dstp_ref[0:1, :E] = ei[1:2, :]
    iot = lax.broadcasted_iota(jnp.int32, (1, E_PAD - E), 1)
    dstp_ref[0:1, E:] = N + (iot % 16)


def _idx_prep(edge_index):
    src2, dstp = pl.pallas_call(
        _idx_prep_body,
        out_shape=[
            jax.ShapeDtypeStruct((2, E_PAD), jnp.int32),
            jax.ShapeDtypeStruct((1, E_PAD), jnp.int32),
        ],
    )(edge_index)
    return src2, dstp.reshape(NCHPAD, CHUNK)


def _mlp_body(h_ref, num_ref, den_ref, w1_ref, b1_ref, g1_ref, be_ref,
              w2_ref, b2_ref, hn_ref, tab_ref):
    numer = jnp.concatenate([num_ref[:N, :], num_ref[N:, :]], axis=1)
    denom = jnp.concatenate([den_ref[:N, :], den_ref[N:, :]], axis=1)
    h = h_ref[...]
    h2 = h + numer / (denom + 1e-16)
    m = jnp.dot(h2, w1_ref[...], preferred_element_type=jnp.float32)
    m = m + b1_ref[...]
    mu = jnp.mean(m, axis=0)
    var = jnp.mean((m - mu) ** 2, axis=0)
    m = g1_ref[...] * (m - mu) / jnp.sqrt(var + 1e-5) + be_ref[...]
    m = jnp.maximum(m, 0.0)
    m = jnp.dot(m, w2_ref[...], preferred_element_type=jnp.float32)
    m = m + b2_ref[...]
    hn = m + h
    hn_ref[...] = hn
    tab_ref[:N, :] = hn[:, :HD]
    tab_ref[N:, :] = hn[:, HD:]


def _mlp_layer(h, num, den, w1, b1, g1, be, w2, b2):
    return pl.pallas_call(
        _mlp_body,
        out_shape=[
            jax.ShapeDtypeStruct((N, D), jnp.float32),
            jax.ShapeDtypeStruct((2 * N, HD), jnp.float32),
        ],
    )(h, num, den, w1, b1, g1, be, w2, b2)


def _pool_body(h_ref, batch_ref, wo_ref, bo_ref, out_ref):
    ids = lax.broadcasted_iota(jnp.int32, (G, N), 0)
    onehot = (batch_ref[...] == ids).astype(jnp.float32)
    counts = jnp.sum(onehot, axis=1, keepdims=True)
    sums = jnp.dot(onehot, h_ref[...], preferred_element_type=jnp.float32)
    pooled = sums / jnp.maximum(counts, 1.0)
    z = jnp.dot(pooled, wo_ref[...], preferred_element_type=jnp.float32)
    out_ref[...] = jax.nn.sigmoid(z + bo_ref[...])


def _pool_head(h, batch, wo, bo):
    return pl.pallas_call(
        _pool_body,
        out_shape=jax.ShapeDtypeStruct((G, 1), jnp.float32),
    )(h, batch, wo, bo)


def _sc_edge_body(h_tab, e_tab, src_hbm, dst_hbm, t_hbm, num_out, den_out,
                  idx_src, idx_dst, hbuf0, ebuf0, hbuf1, ebuf1, tbuf,
                  acc_n, acc_d, sg0, se0, ss0, sg1, se1, ss1):
    c = lax.axis_index("c")
    s = lax.axis_index("s")
    start = s * CPT

    pltpu.sync_copy(t_hbm, tbuf)
    tval = tbuf[...]

    def _z(r, _):
        for k in range(HD // 16):
            hbuf0[r, pl.ds(k * 16, 16)] = jnp.zeros((16,), jnp.float32)
        return 0

    lax.fori_loop(0, CHUNK, _z, 0)
    base = s * NPT_A
    for acc in (acc_n, acc_d):
        for j in range(4):
            pltpu.sync_copy(hbuf0, acc.at[pl.ds(base + j * CHUNK, CHUNK), :])

    @pl.when(s < NSUB - 1)
    def _():
        for acc in (acc_n, acc_d):
            pltpu.sync_copy(hbuf0.at[pl.ds(0, NPT_A - 512), :],
                            acc.at[pl.ds(base + 512, NPT_A - 512), :])

    @pl.when(s == NSUB - 1)
    def _():
        for acc in (acc_n, acc_d):
            pltpu.sync_copy(hbuf0.at[pl.ds(0, NPT_ZL - 512), :],
                            acc.at[pl.ds(base + 512, NPT_ZL - 512), :])

    plsc.subcore_barrier()

    hb = (hbuf0, hbuf1)
    eb = (ebuf0, ebuf1)
    sg = (sg0, sg1)
    se = (se0, se1)
    ss = (ss0, ss1)

    def _blk(b, _):
        blk0 = start + b * CBLK
        pltpu.sync_copy(src_hbm.at[c, pl.ds(blk0 * CHUNK, CBLK * CHUNK)],
                        idx_src)
        pltpu.sync_copy(dst_hbm.at[pl.ds(blk0, CBLK), :], idx_dst)

        def _issue(j):
            sl = j % 2
            cg = pltpu.async_copy(
                h_tab.at[idx_src.at[pl.ds(j * CHUNK, CHUNK)]], hb[sl],
                sg[sl])
            ce = pltpu.async_copy(
                e_tab.at[pl.ds(c * E_PAD + (blk0 + j) * CHUNK, CHUNK), :],
                eb[sl], se[sl])
            return cg, ce

        loads = [None] * CBLK
        scats = [None] * CBLK
        loads[0] = _issue(0)
        for j in range(CBLK):
            sl = j % 2
            if j >= 1:
                sn, sd = scats[j - 1]
                sn.wait()
                sd.wait()
            if j + 1 < CBLK:
                loads[j + 1] = _issue(j + 1)
            cg, ce = loads[j]
            cg.wait()
            ce.wait()

            @plsc.parallel_loop(0, CHUNK, unroll=4)
            def _row(r):
                for k in range(HD // 16):
                    hv = hb[sl][r, pl.ds(k * 16, 16)]
                    ev = eb[sl][r, pl.ds(k * 16, 16)]
                    msg = jnp.maximum(hv + ev, 0.0) + 1e-7
                    w = jnp.exp(msg * tval)
                    hb[sl][r, pl.ds(k * 16, 16)] = msg * w
                    eb[sl][r, pl.ds(k * 16, 16)] = w
            sn = pltpu.async_copy(hb[sl], acc_n.at[idx_dst.at[j]], ss[sl],
                                  add=True)
            sd = pltpu.async_copy(eb[sl], acc_d.at[idx_dst.at[j]], ss[sl],
                                  add=True)
            scats[j] = (sn, sd)
        for j in (CBLK - 1,):
            sn, sd = scats[j]
            sn.wait()
            sd.wait()
        return 0

    lax.fori_loop(0, CPT // CBLK, _blk, 0)
    plsc.subcore_barrier()

    out0 = c * N + base
    for acc, out in ((acc_n, num_out), (acc_d, den_out)):
        for j in range(4):
            pltpu.sync_copy(acc.at[pl.ds(base + j * CHUNK, CHUNK), :],
                            out.at[pl.ds(out0 + j * CHUNK, CHUNK), :])

    @pl.when(s < NSUB - 1)
    def _():
        for acc, out in ((acc_n, num_out), (acc_d, den_out)):
            pltpu.sync_copy(acc.at[pl.ds(base + 512, NPT_A - 512), :],
                            out.at[pl.ds(out0 + 512, NPT_A - 512), :])

    @pl.when(s == NSUB - 1)
    def _():
        for acc, out in ((acc_n, num_out), (acc_d, den_out)):
            pltpu.sync_copy(acc.at[pl.ds(base + 512, NPT_SL - 512), :],
                            out.at[pl.ds(out0 + 512, NPT_SL - 512), :])


def _sc_edge_pass(h_tab, e_tab, src, dst3, t16):
    mesh = plsc.VectorSubcoreMesh(core_axis_name="c", subcore_axis_name="s")
    k = functools.partial(
        pl.kernel,
        out_type=[
            jax.ShapeDtypeStruct((2 * N, HD), jnp.float32),
            jax.ShapeDtypeStruct((2 * N, HD), jnp.float32),
        ],
        mesh=mesh,
        compiler_params=pltpu.CompilerParams(use_tc_tiling_on_sc=False),
        scratch_types=[
            pltpu.VMEM((CBLK * CHUNK,), jnp.int32),
            pltpu.VMEM((CBLK, CHUNK), jnp.int32),
            pltpu.VMEM((CHUNK, HD), jnp.float32),
            pltpu.VMEM((CHUNK, HD), jnp.float32),
            pltpu.VMEM((CHUNK, HD), jnp.float32),
            pltpu.VMEM((CHUNK, HD), jnp.float32),
            pltpu.VMEM((16,), jnp.float32),
            pltpu.VMEM_SHARED((NROW, HD), jnp.float32),
            pltpu.VMEM_SHARED((NROW, HD), jnp.float32),
            pltpu.SemaphoreType.DMA,
            pltpu.SemaphoreType.DMA,
            pltpu.SemaphoreType.DMA,
            pltpu.SemaphoreType.DMA,
            pltpu.SemaphoreType.DMA,
            pltpu.SemaphoreType.DMA,
        ],
    )(_sc_edge_body)
    return k(h_tab, e_tab, src, dst3, t16)


def kernel(x, edge_index, edge_attr, batch, data, W_node, b_node, W_edge,
           b_edge, t, W1, b1, g1, beta1, W2, b2, W_out, b_out):
    h, h_tab = _node_lin(x, W_node, b_node.reshape(1, D))
    e_tab = _edge_lin(edge_attr, W_edge, b_edge.reshape(1, D))
    src, dst3 = _idx_prep(edge_index)
    for i in range(L):
        t16 = jnp.broadcast_to(t[i], (16,))
        num, den = _sc_edge_pass(h_tab, e_tab, src, dst3, t16)
        h, h_tab = _mlp_layer(h, num, den, W1[i], b1[i].reshape(1, H),
                              g1[i].reshape(1, H), beta1[i].reshape(1, H),
                              W2[i], b2[i].reshape(1, D))
    return _pool_head(h, batch.reshape(1, N), W_out, b_out.reshape(1, 1))


# speedup vs baseline: 1.1518x; 1.1518x over previous
import functools

import jax
import jax.numpy as jnp
from jax import lax
from jax.experimental import pallas as pl
from jax.experimental.pallas import tpu as pltpu
from jax.experimental.pallas import tpu_sc as plsc

N = 10000
E = 320000
D = 128
DE = 16
H = 256
L = 4
G = 64

HD = D // 2
CHUNK = 128
NCHUNK = E // CHUNK
NSUB = 16
CPT = 160
CBLK = 16
NCHPAD = CPT * NSUB
E_PAD = NCHPAD * CHUNK
NROW = N + 16
NPT_A = 632
NPT_ZL = NROW - 15 * NPT_A
NPT_SL = N - 15 * NPT_A


def _node_lin_body(x_ref, w_ref, b_ref, h_ref, tab_ref):
    h = jnp.dot(x_ref[...], w_ref[...], preferred_element_type=jnp.float32)
    h = h + b_ref[...]
    h_ref[...] = h
    tab_ref[:N, :] = h[:, :HD]
    tab_ref[N:, :] = h[:, HD:]


def _node_lin(x, w, b):
    return pl.pallas_call(
        _node_lin_body,
        out_shape=[
            jax.ShapeDtypeStruct((N, D), jnp.float32),
            jax.ShapeDtypeStruct((2 * N, HD), jnp.float32),
        ],
    )(x, w, b)


_BE = 16384


def _edge_lin_body(a_ref, w_ref, b_ref, out_ref):
    e = jnp.dot(a_ref[...], w_ref[...], preferred_element_type=jnp.float32)
    e = e + b_ref[...]
    out_ref[0] = e[:, :HD]
    out_ref[1] = e[:, HD:]


def _edge_lin(edge_attr, w, b):
    out = pl.pallas_call(
        _edge_lin_body,
        grid=(E_PAD // _BE,),
        in_specs=[
            pl.BlockSpec((_BE, DE), lambda i: (i, 0)),
            pl.BlockSpec((DE, D), lambda i: (0, 0)),
            pl.BlockSpec((1, D), lambda i: (0, 0)),
        ],
        out_specs=pl.BlockSpec((2, _BE, HD), lambda i: (0, i, 0)),
        out_shape=jax.ShapeDtypeStruct((2, E_PAD, HD), jnp.float32),
    )(edge_attr, w, b)
    return out.reshape(2 * E_PAD, HD)


def _idx_prep(edge_index):
    npad = E_PAD - E
    src = jnp.concatenate([edge_index[0], jnp.zeros((npad,), jnp.int32)])
    src2 = jnp.stack([src, src + N])
    dst3 = jnp.concatenate(
        [edge_index[1],
         N + (jnp.arange(npad, dtype=jnp.int32) % 16)]).reshape(NCHPAD, CHUNK)
    return src2, dst3


def _mlp_body(h_ref, num_ref, den_ref, w1_ref, b1_ref, g1_ref, be_ref,
              w2_ref, b2_ref, hn_ref, tab_ref):
    numer = jnp.concatenate([num_ref[:N, :], num_ref[N:, :]], axis=1)
    denom = jnp.concatenate([den_ref[:N, :], den_ref[N:, :]], axis=1)
    h = h_ref[...]
    h2 = h + numer / (denom + 1e-16)
    m = jnp.dot(h2, w1_ref[...], preferred_element_type=jnp.float32)
    m = m + b1_ref[...]
    mu = jnp.mean(m, axis=0)
    var = jnp.mean((m - mu) ** 2, axis=0)
    m = g1_ref[...] * (m - mu) / jnp.sqrt(var + 1e-5) + be_ref[...]
    m = jnp.maximum(m, 0.0)
    m = jnp.dot(m, w2_ref[...], preferred_element_type=jnp.float32)
    m = m + b2_ref[...]
    hn = m + h
    hn_ref[...] = hn
    tab_ref[:N, :] = hn[:, :HD]
    tab_ref[N:, :] = hn[:, HD:]


def _mlp_layer(h, num, den, w1, b1, g1, be, w2, b2):
    return pl.pallas_call(
        _mlp_body,
        out_shape=[
            jax.ShapeDtypeStruct((N, D), jnp.float32),
            jax.ShapeDtypeStruct((2 * N, HD), jnp.float32),
        ],
    )(h, num, den, w1, b1, g1, be, w2, b2)


def _pool_body(h_ref, batch_ref, wo_ref, bo_ref, out_ref):
    ids = lax.broadcasted_iota(jnp.int32, (G, N), 0)
    onehot = (batch_ref[...] == ids).astype(jnp.float32)
    counts = jnp.sum(onehot, axis=1, keepdims=True)
    sums = jnp.dot(onehot, h_ref[...], preferred_element_type=jnp.float32)
    pooled = sums / jnp.maximum(counts, 1.0)
    z = jnp.dot(pooled, wo_ref[...], preferred_element_type=jnp.float32)
    out_ref[...] = jax.nn.sigmoid(z + bo_ref[...])


def _pool_head(h, batch, wo, bo):
    return pl.pallas_call(
        _pool_body,
        out_shape=jax.ShapeDtypeStruct((G, 1), jnp.float32),
    )(h, batch, wo, bo)


def _sc_edge_body(h_tab, e_tab, src_hbm, dst_hbm, t_hbm, num_out, den_out,
                  idx_src, idx_dst, hbuf0, ebuf0, hbuf1, ebuf1, tbuf,
                  acc_n, acc_d, sg0, se0, ss0, sg1, se1, ss1):
    c = lax.axis_index("c")
    s = lax.axis_index("s")
    start = s * CPT

    pltpu.sync_copy(t_hbm, tbuf)
    tval = tbuf[...]

    def _z(r, _):
        for k in range(HD // 16):
            hbuf0[r, pl.ds(k * 16, 16)] = jnp.zeros((16,), jnp.float32)
        return 0

    lax.fori_loop(0, CHUNK, _z, 0)
    base = s * NPT_A
    for acc in (acc_n, acc_d):
        for j in range(4):
            pltpu.sync_copy(hbuf0, acc.at[pl.ds(base + j * CHUNK, CHUNK), :])

    @pl.when(s < NSUB - 1)
    def _():
        for acc in (acc_n, acc_d):
            pltpu.sync_copy(hbuf0.at[pl.ds(0, NPT_A - 512), :],
                            acc.at[pl.ds(base + 512, NPT_A - 512), :])

    @pl.when(s == NSUB - 1)
    def _():
        for acc in (acc_n, acc_d):
            pltpu.sync_copy(hbuf0.at[pl.ds(0, NPT_ZL - 512), :],
                            acc.at[pl.ds(base + 512, NPT_ZL - 512), :])

    plsc.subcore_barrier()

    hb = (hbuf0, hbuf1)
    eb = (ebuf0, ebuf1)
    sg = (sg0, sg1)
    se = (se0, se1)
    ss = (ss0, ss1)

    def _blk(b, _):
        blk0 = start + b * CBLK
        pltpu.sync_copy(src_hbm.at[c, pl.ds(blk0 * CHUNK, CBLK * CHUNK)],
                        idx_src)
        pltpu.sync_copy(dst_hbm.at[pl.ds(blk0, CBLK), :], idx_dst)

        def _issue(j):
            sl = j % 2
            cg = pltpu.async_copy(
                h_tab.at[idx_src.at[pl.ds(j * CHUNK, CHUNK)]], hb[sl],
                sg[sl])
            ce = pltpu.async_copy(
                e_tab.at[pl.ds(c * E_PAD + (blk0 + j) * CHUNK, CHUNK), :],
                eb[sl], se[sl])
            return cg, ce

        loads = [None] * CBLK
        scats = [None] * CBLK
        loads[0] = _issue(0)
        for j in range(CBLK):
            sl = j % 2
            if j >= 1:
                sn, sd = scats[j - 1]
                sn.wait()
                sd.wait()
            if j + 1 < CBLK:
                loads[j + 1] = _issue(j + 1)
            cg, ce = loads[j]
            cg.wait()
            ce.wait()

            @plsc.parallel_loop(0, CHUNK, unroll=4)
            def _row(r):
                for k in range(HD // 16):
                    hv = hb[sl][r, pl.ds(k * 16, 16)]
                    ev = eb[sl][r, pl.ds(k * 16, 16)]
                    msg = jnp.maximum(hv + ev, 0.0) + 1e-7
                    w = jnp.exp(msg * tval)
                    hb[sl][r, pl.ds(k * 16, 16)] = msg * w
                    eb[sl][r, pl.ds(k * 16, 16)] = w
            sn = pltpu.async_copy(hb[sl], acc_n.at[idx_dst.at[j]], ss[sl],
                                  add=True)
            sd = pltpu.async_copy(eb[sl], acc_d.at[idx_dst.at[j]], ss[sl],
                                  add=True)
            scats[j] = (sn, sd)
        for j in (CBLK - 1,):
            sn, sd = scats[j]
            sn.wait()
            sd.wait()
        return 0

    lax.fori_loop(0, CPT // CBLK, _blk, 0)
    plsc.subcore_barrier()

    out0 = c * N + base
    for acc, out in ((acc_n, num_out), (acc_d, den_out)):
        for j in range(4):
            pltpu.sync_copy(acc.at[pl.ds(base + j * CHUNK, CHUNK), :],
                            out.at[pl.ds(out0 + j * CHUNK, CHUNK), :])

    @pl.when(s < NSUB - 1)
    def _():
        for acc, out in ((acc_n, num_out), (acc_d, den_out)):
            pltpu.sync_copy(acc.at[pl.ds(base + 512, NPT_A - 512), :],
                            out.at[pl.ds(out0 + 512, NPT_A - 512), :])

    @pl.when(s == NSUB - 1)
    def _():
        for acc, out in ((acc_n, num_out), (acc_d, den_out)):
            pltpu.sync_copy(acc.at[pl.ds(base + 512, NPT_SL - 512), :],
                            out.at[pl.ds(out0 + 512, NPT_SL - 512), :])


def _sc_edge_pass(h_tab, e_tab, src, dst3, t16):
    mesh = plsc.VectorSubcoreMesh(core_axis_name="c", subcore_axis_name="s")
    k = functools.partial(
        pl.kernel,
        out_type=[
            jax.ShapeDtypeStruct((2 * N, HD), jnp.float32),
            jax.ShapeDtypeStruct((2 * N, HD), jnp.float32),
        ],
        mesh=mesh,
        compiler_params=pltpu.CompilerParams(use_tc_tiling_on_sc=False),
        scratch_types=[
            pltpu.VMEM((CBLK * CHUNK,), jnp.int32),
            pltpu.VMEM((CBLK, CHUNK), jnp.int32),
            pltpu.VMEM((CHUNK, HD), jnp.float32),
            pltpu.VMEM((CHUNK, HD), jnp.float32),
            pltpu.VMEM((CHUNK, HD), jnp.float32),
            pltpu.VMEM((CHUNK, HD), jnp.float32),
            pltpu.VMEM((16,), jnp.float32),
            pltpu.VMEM_SHARED((NROW, HD), jnp.float32),
            pltpu.VMEM_SHARED((NROW, HD), jnp.float32),
            pltpu.SemaphoreType.DMA,
            pltpu.SemaphoreType.DMA,
            pltpu.SemaphoreType.DMA,
            pltpu.SemaphoreType.DMA,
            pltpu.SemaphoreType.DMA,
            pltpu.SemaphoreType.DMA,
        ],
    )(_sc_edge_body)
    return k(h_tab, e_tab, src, dst3, t16)


def kernel(x, edge_index, edge_attr, batch, data, W_node, b_node, W_edge,
           b_edge, t, W1, b1, g1, beta1, W2, b2, W_out, b_out):
    h, h_tab = _node_lin(x, W_node, b_node.reshape(1, D))
    e_tab = _edge_lin(edge_attr, W_edge, b_edge.reshape(1, D))
    src, dst3 = _idx_prep(edge_index)
    for i in range(L):
        t16 = jnp.broadcast_to(t[i], (16,))
        num, den = _sc_edge_pass(h_tab, e_tab, src, dst3, t16)
        h, h_tab = _mlp_layer(h, num, den, W1[i], b1[i].reshape(1, H),
                              g1[i].reshape(1, H), beta1[i].reshape(1, H),
                              W2[i], b2[i].reshape(1, D))
    return _pool_head(h, batch.reshape(1, N), W_out, b_out.reshape(1, 1))


# speedup vs baseline: 1.4176x; 1.2308x over previous
import functools

import jax
import jax.numpy as jnp
from jax import lax
from jax.experimental import pallas as pl
from jax.experimental.pallas import tpu as pltpu
from jax.experimental.pallas import tpu_sc as plsc

N = 10000
E = 320000
D = 128
DE = 16
H = 256
L = 4
G = 64

HD = D // 2
CHUNK = 120
NSUB = 16
CPT = 168
CBLK = 12
NCHPAD = CPT * NSUB
E_PAD = NCHPAD * CHUNK
NROW = N + 8
NPT_A = 632
NPT_ZL = NROW - 15 * NPT_A
NPT_SL = N - 15 * NPT_A


def _row_pieces(n):
    sizes = [CHUNK] * (n // CHUNK)
    if n % CHUNK:
        sizes.append(n % CHUNK)
    offs, o = [], 0
    for sz in sizes:
        offs.append(o)
        o += sz
    return list(zip(offs, sizes))


def _node_lin_body(x_ref, w_ref, b_ref, h_ref, tab_ref):
    h = jnp.dot(x_ref[...], w_ref[...], preferred_element_type=jnp.float32)
    h = h + b_ref[...]
    h_ref[...] = h
    tab_ref[:N, :] = h[:, :HD]
    tab_ref[N:, :] = h[:, HD:]


def _node_lin(x, w, b):
    return pl.pallas_call(
        _node_lin_body,
        out_shape=[
            jax.ShapeDtypeStruct((N, D), jnp.float32),
            jax.ShapeDtypeStruct((2 * N, HD), jnp.float32),
        ],
    )(x, w, b)


_BE = 16128


def _edge_lin_body(a_ref, w_ref, b_ref, out_ref):
    e = jnp.dot(a_ref[...], w_ref[...], preferred_element_type=jnp.float32)
    e = e + b_ref[...]
    out_ref[0] = e[:, :HD]
    out_ref[1] = e[:, HD:]


def _edge_lin(edge_attr, w, b):
    out = pl.pallas_call(
        _edge_lin_body,
        grid=(E_PAD // _BE,),
        in_specs=[
            pl.BlockSpec((_BE, DE), lambda i: (i, 0)),
            pl.BlockSpec((DE, D), lambda i: (0, 0)),
            pl.BlockSpec((1, D), lambda i: (0, 0)),
        ],
        out_specs=pl.BlockSpec((2, _BE, HD), lambda i: (0, i, 0)),
        out_shape=jax.ShapeDtypeStruct((2, E_PAD, HD), jnp.float32),
    )(edge_attr, w, b)
    return out.reshape(2 * E_PAD, HD)


def _idx_prep(edge_index):
    npad = E_PAD - E
    src = jnp.concatenate([edge_index[0], jnp.zeros((npad,), jnp.int32)])
    src2 = jnp.stack([src, src + N])
    dst3 = jnp.concatenate(
        [edge_index[1],
         N + (jnp.arange(npad, dtype=jnp.int32) % 8)]).reshape(NCHPAD, CHUNK)
    return src2, dst3


def _mlp_body(h_ref, num_ref, den_ref, w1_ref, b1_ref, g1_ref, be_ref,
              w2_ref, b2_ref, hn_ref, tab_ref):
    numer = jnp.concatenate([num_ref[:N, :], num_ref[N:, :]], axis=1)
    denom = jnp.concatenate([den_ref[:N, :], den_ref[N:, :]], axis=1)
    h = h_ref[...]
    h2 = h + numer / (denom + 1e-16)
    m = jnp.dot(h2, w1_ref[...], preferred_element_type=jnp.float32)
    m = m + b1_ref[...]
    mu = jnp.mean(m, axis=0)
    var = jnp.mean((m - mu) ** 2, axis=0)
    m = g1_ref[...] * (m - mu) / jnp.sqrt(var + 1e-5) + be_ref[...]
    m = jnp.maximum(m, 0.0)
    m = jnp.dot(m, w2_ref[...], preferred_element_type=jnp.float32)
    m = m + b2_ref[...]
    hn = m + h
    hn_ref[...] = hn
    tab_ref[:N, :] = hn[:, :HD]
    tab_ref[N:, :] = hn[:, HD:]


def _mlp_layer(h, num, den, w1, b1, g1, be, w2, b2):
    return pl.pallas_call(
        _mlp_body,
        out_shape=[
            jax.ShapeDtypeStruct((N, D), jnp.float32),
            jax.ShapeDtypeStruct((2 * N, HD), jnp.float32),
        ],
    )(h, num, den, w1, b1, g1, be, w2, b2)


def _pool_body(h_ref, batch_ref, wo_ref, bo_ref, out_ref):
    ids = lax.broadcasted_iota(jnp.int32, (G, N), 0)
    onehot = (batch_ref[...] == ids).astype(jnp.float32)
    counts = jnp.sum(onehot, axis=1, keepdims=True)
    sums = jnp.dot(onehot, h_ref[...], preferred_element_type=jnp.float32)
    pooled = sums / jnp.maximum(counts, 1.0)
    z = jnp.dot(pooled, wo_ref[...], preferred_element_type=jnp.float32)
    out_ref[...] = jax.nn.sigmoid(z + bo_ref[...])


def _pool_head(h, batch, wo, bo):
    return pl.pallas_call(
        _pool_body,
        out_shape=jax.ShapeDtypeStruct((G, 1), jnp.float32),
    )(h, batch, wo, bo)


def _sc_edge_body(h_tab, e_tab, src_hbm, dst_hbm, t_hbm, num_out, den_out,
                  idx_src, idx_dst, hbuf0, ebuf0, hbuf1, ebuf1, hbuf2, ebuf2,
                  tbuf, acc_n, acc_d, sg0, se0, ss0, sg1, se1, ss1, sg2, se2,
                  ss2):
    c = lax.axis_index("c")
    s = lax.axis_index("s")
    start = s * CPT

    pltpu.sync_copy(t_hbm, tbuf)
    tval = tbuf[...]

    def _z(r, _):
        for k in range(HD // 16):
            hbuf0[r, pl.ds(k * 16, 16)] = jnp.zeros((16,), jnp.float32)
        return 0

    lax.fori_loop(0, CHUNK, _z, 0)
    base = s * NPT_A

    @pl.when(s < NSUB - 1)
    def _():
        for acc in (acc_n, acc_d):
            for off, sz in _row_pieces(NPT_A):
                pltpu.sync_copy(hbuf0.at[pl.ds(0, sz), :],
                                acc.at[pl.ds(base + off, sz), :])

    @pl.when(s == NSUB - 1)
    def _():
        for acc in (acc_n, acc_d):
            for off, sz in _row_pieces(NPT_ZL):
                pltpu.sync_copy(hbuf0.at[pl.ds(0, sz), :],
                                acc.at[pl.ds(base + off, sz), :])

    plsc.subcore_barrier()

    hb = (hbuf0, hbuf1, hbuf2)
    eb = (ebuf0, ebuf1, ebuf2)
    sg = (sg0, sg1, sg2)
    se = (se0, se1, se2)
    ss = (ss0, ss1, ss2)

    def _blk(b, _):
        blk0 = start + b * CBLK
        pltpu.sync_copy(src_hbm.at[c, pl.ds(blk0 * CHUNK, CBLK * CHUNK)],
                        idx_src)
        pltpu.sync_copy(dst_hbm.at[pl.ds(blk0, CBLK), :], idx_dst)

        def _issue(j):
            sl = j % 3
            cg = pltpu.async_copy(
                h_tab.at[idx_src.at[pl.ds(j * CHUNK, CHUNK)]], hb[sl],
                sg[sl])
            ce = pltpu.async_copy(
                e_tab.at[pl.ds(c * E_PAD + (blk0 + j) * CHUNK, CHUNK), :],
                eb[sl], se[sl])
            return cg, ce

        loads = [None] * CBLK
        scats = [None] * CBLK
        loads[0] = _issue(0)
        for j in range(CBLK):
            sl = j % 3
            if j >= 2:
                sn, sd = scats[j - 2]
                sn.wait()
                sd.wait()
            if j + 1 < CBLK:
                loads[j + 1] = _issue(j + 1)
            cg, ce = loads[j]
            cg.wait()
            ce.wait()

            @plsc.parallel_loop(0, CHUNK, unroll=4)
            def _row(r):
                for k in range(HD // 16):
                    hv = hb[sl][r, pl.ds(k * 16, 16)]
                    ev = eb[sl][r, pl.ds(k * 16, 16)]
                    msg = jnp.maximum(hv + ev, 0.0) + 1e-7
                    w = jnp.exp(msg * tval)
                    hb[sl][r, pl.ds(k * 16, 16)] = msg * w
                    eb[sl][r, pl.ds(k * 16, 16)] = w
            sn = pltpu.async_copy(hb[sl], acc_n.at[idx_dst.at[j]], ss[sl],
                                  add=True)
            sd = pltpu.async_copy(eb[sl], acc_d.at[idx_dst.at[j]], ss[sl],
                                  add=True)
            scats[j] = (sn, sd)
        for j in (CBLK - 2, CBLK - 1):
            sn, sd = scats[j]
            sn.wait()
            sd.wait()
        return 0

    lax.fori_loop(0, CPT // CBLK, _blk, 0)
    plsc.subcore_barrier()

    out0 = c * N + base

    @pl.when(s < NSUB - 1)
    def _():
        for acc, out in ((acc_n, num_out), (acc_d, den_out)):
            for off, sz in _row_pieces(NPT_A):
                pltpu.sync_copy(acc.at[pl.ds(base + off, sz), :],
                                out.at[pl.ds(out0 + off, sz), :])

    @pl.when(s == NSUB - 1)
    def _():
        for acc, out in ((acc_n, num_out), (acc_d, den_out)):
            for off, sz in _row_pieces(NPT_SL):
                pltpu.sync_copy(acc.at[pl.ds(base + off, sz), :],
                                out.at[pl.ds(out0 + off, sz), :])


def _sc_edge_pass(h_tab, e_tab, src, dst3, t16):
    mesh = plsc.VectorSubcoreMesh(core_axis_name="c", subcore_axis_name="s")
    k = functools.partial(
        pl.kernel,
        out_type=[
            jax.ShapeDtypeStruct((2 * N, HD), jnp.float32),
            jax.ShapeDtypeStruct((2 * N, HD), jnp.float32),
        ],
        mesh=mesh,
        compiler_params=pltpu.CompilerParams(use_tc_tiling_on_sc=False),
        scratch_types=[
            pltpu.VMEM((CBLK * CHUNK,), jnp.int32),
            pltpu.VMEM((CBLK, CHUNK), jnp.int32),
            pltpu.VMEM((CHUNK, HD), jnp.float32),
            pltpu.VMEM((CHUNK, HD), jnp.float32),
            pltpu.VMEM((CHUNK, HD), jnp.float32),
            pltpu.VMEM((CHUNK, HD), jnp.float32),
            pltpu.VMEM((CHUNK, HD), jnp.float32),
            pltpu.VMEM((CHUNK, HD), jnp.float32),
            pltpu.VMEM((16,), jnp.float32),
            pltpu.VMEM_SHARED((NROW, HD), jnp.float32),
            pltpu.VMEM_SHARED((NROW, HD), jnp.float32),
            pltpu.SemaphoreType.DMA,
            pltpu.SemaphoreType.DMA,
            pltpu.SemaphoreType.DMA,
            pltpu.SemaphoreType.DMA,
            pltpu.SemaphoreType.DMA,
            pltpu.SemaphoreType.DMA,
            pltpu.SemaphoreType.DMA,
            pltpu.SemaphoreType.DMA,
            pltpu.SemaphoreType.DMA,
        ],
    )(_sc_edge_body)
    return k(h_tab, e_tab, src, dst3, t16)


def kernel(x, edge_index, edge_attr, batch, data, W_node, b_node, W_edge,
           b_edge, t, W1, b1, g1, beta1, W2, b2, W_out, b_out):
    h, h_tab = _node_lin(x, W_node, b_node.reshape(1, D))
    e_tab = _edge_lin(edge_attr, W_edge, b_edge.reshape(1, D))
    src, dst3 = _idx_prep(edge_index)
    for i in range(L):
        t16 = jnp.broadcast_to(t[i], (16,))
        num, den = _sc_edge_pass(h_tab, e_tab, src, dst3, t16)
        h, h_tab = _mlp_layer(h, num, den, W1[i], b1[i].reshape(1, H),
                              g1[i].reshape(1, H), beta1[i].reshape(1, H),
                              W2[i], b2[i].reshape(1, D))
    return _pool_head(h, batch.reshape(1, N), W_out, b_out.reshape(1, 1))
